# Initial kernel scaffold; baseline (speedup 1.0000x reference)
import functools

import jax
import jax.numpy as jnp
from jax import lax
from jax.experimental import pallas as pl
from jax.experimental.pallas import tpu as pltpu
from jax.experimental.pallas import tpu_sc as plsc

N_NODES = 10000
N_EDGES = 320000
NPAD = 10240
NC, NS, L = 2, 16, 16

E_PER_SUB = N_EDGES // NS
EPB = 128
N_FULL_BLK = E_PER_SUB // EPB
EPB_TAIL = E_PER_SUB - N_FULL_BLK * EPB

ROWS_PER_SUB = N_NODES // NS

_MESH = plsc.VectorSubcoreMesh(
    core_axis_name="c", subcore_axis_name="s", num_cores=NC, num_subcores=NS)


def _rsqrt16(d):
    i = plsc.bitcast(d, jnp.int32)
    i = 0x5F3759DF - lax.shift_right_logical(i, 1)
    y = plsc.bitcast(i, jnp.float32)
    for _ in range(3):
        y = y * (1.5 - 0.5 * d * y * y)
    return y


_DEG_CHUNK = 2000


def _deg_body(dst_hbm, dinv_hbm, hist, dbuf, slab, obuf, sem):
    c = lax.axis_index("c")
    s = lax.axis_index("s")

    def zero_body(i, carry):
        hist[pl.ds(i * L, L)] = jnp.zeros((L,), jnp.float32)
        return carry
    lax.fori_loop(0, NPAD // L, zero_body, 0)

    ones = jnp.ones((L,), jnp.float32)

    def chunk_body(j, carry):
        pltpu.sync_copy(dst_hbm.at[pl.ds(s * E_PER_SUB + j * _DEG_CHUNK,
                                         _DEG_CHUNK)], dbuf)

        def vec_body(k, carry2):
            idx = dbuf[pl.ds(k * L, L)]
            plsc.addupdate_scatter(hist, [idx], ones)
            return carry2
        return lax.fori_loop(0, _DEG_CHUNK // L, vec_body, carry)
    lax.fori_loop(0, E_PER_SUB // _DEG_CHUNK, chunk_body, 0)

    pltpu.sync_copy(hist, slab.at[s])
    plsc.subcore_barrier()

    span = NPAD // (NC * NS)
    g0 = c * (NPAD // NC) + s * span

    def red_body(k, carry):
        base = g0 + k * L
        tot = jnp.ones((L,), jnp.float32)
        for j in range(NS):
            tot = tot + slab[j, pl.ds(base, L)]
        obuf[pl.ds(k * L, L)] = _rsqrt16(tot)
        return carry
    lax.fori_loop(0, span // L, red_body, 0)

    pltpu.sync_copy(obuf, dinv_hbm.at[pl.ds(g0, span)])


def _make_deg_kernel():
    return pl.kernel(
        _deg_body,
        out_type=jax.ShapeDtypeStruct((NPAD,), jnp.float32),
        mesh=_MESH,
        scratch_types=[
            pltpu.VMEM((NPAD,), jnp.float32),
            pltpu.VMEM((_DEG_CHUNK,), jnp.int32),
            pltpu.VMEM_SHARED((NS, NPAD), jnp.float32),
            pltpu.VMEM((NPAD // (NC * NS),), jnp.float32),
            pltpu.SemaphoreType.DMA,
        ],
    )


def _agg_body(hd, hp_hbm, src_hbm, dst_hbm, out_hbm,
              sbuf, dbuf, sbuf_t, dbuf_t, rows, acc, sem):
    c = lax.axis_index("c")
    s = lax.axis_index("s")
    c_off = c * N_NODES

    def zfill(i, carry):
        for j in range(hd // L):
            rows[i, pl.ds(j * L, L)] = jnp.zeros((L,), jnp.float32)
        return carry
    lax.fori_loop(0, EPB, zfill, 0)

    r0 = s * ROWS_PER_SUB
    n_zfull = ROWS_PER_SUB // EPB
    z_tail = ROWS_PER_SUB - n_zfull * EPB
    for k in range(n_zfull):
        pltpu.sync_copy(rows, acc.at[pl.ds(r0 + k * EPB, EPB)])
    pltpu.sync_copy(rows.at[pl.ds(0, z_tail)],
                    acc.at[pl.ds(r0 + n_zfull * EPB, z_tail)])
    plsc.subcore_barrier()

    e0 = s * E_PER_SUB

    def blk_body(i, carry):
        off = e0 + i * EPB
        pltpu.sync_copy(src_hbm.at[pl.ds(off, EPB)], sbuf)
        pltpu.sync_copy(dst_hbm.at[pl.ds(off, EPB)], dbuf)
        for j in range(EPB // L):
            sbuf[pl.ds(j * L, L)] = sbuf[pl.ds(j * L, L)] + c_off
        pltpu.async_copy(hp_hbm.at[sbuf], rows, sem).wait()
        pltpu.sync_copy(rows, acc.at[dbuf], add=True)
        return carry
    lax.fori_loop(0, N_FULL_BLK, blk_body, 0)

    t_off = e0 + N_FULL_BLK * EPB
    pltpu.sync_copy(src_hbm.at[pl.ds(t_off, EPB_TAIL)], sbuf_t)
    pltpu.sync_copy(dst_hbm.at[pl.ds(t_off, EPB_TAIL)], dbuf_t)
    for j in range(EPB_TAIL // L):
        sbuf_t[pl.ds(j * L, L)] = sbuf_t[pl.ds(j * L, L)] + c_off
    pltpu.async_copy(hp_hbm.at[sbuf_t], rows.at[pl.ds(0, EPB_TAIL)], sem).wait()
    pltpu.sync_copy(rows.at[pl.ds(0, EPB_TAIL)], acc.at[dbuf_t], add=True)

    plsc.subcore_barrier()

    def wb(k, nrows):
        rr = r0 + k * EPB
        pltpu.sync_copy(acc.at[pl.ds(rr, nrows)], rows.at[pl.ds(0, nrows)])
        pltpu.sync_copy(rows.at[pl.ds(0, nrows)],
                        out_hbm.at[pl.ds(c_off + rr, nrows)])
    for k in range(n_zfull):
        wb(k, EPB)
    wb(n_zfull, z_tail)


def _make_agg_kernel(hd):
    return pl.kernel(
        functools.partial(_agg_body, hd),
        out_type=jax.ShapeDtypeStruct((NC * N_NODES, hd), jnp.float32),
        mesh=_MESH,
        scratch_types=[
            pltpu.VMEM((EPB,), jnp.int32),
            pltpu.VMEM((EPB,), jnp.int32),
            pltpu.VMEM((EPB_TAIL,), jnp.int32),
            pltpu.VMEM((EPB_TAIL,), jnp.int32),
            pltpu.VMEM((EPB, hd), jnp.float32),
            pltpu.VMEM_SHARED((N_NODES, hd), jnp.float32),
            pltpu.SemaphoreType.DMA,
        ],
    )


NB_ROWS = 400
NBLKS = N_NODES // NB_ROWS


def _l1_body(x_ref, w_ref, dinv_ref, out_ref):
    h = jnp.dot(x_ref[...], w_ref[...], preferred_element_type=jnp.float32)
    out_ref[...] = dinv_ref[...] * h


def _layer1(x, w, dinv_col, hd):
    in_dim = x.shape[1]
    return pl.pallas_call(
        _l1_body,
        grid=(NBLKS, NC),
        in_specs=[
            pl.BlockSpec((NB_ROWS, in_dim), lambda i, c: (i, 0)),
            pl.BlockSpec((in_dim, hd), lambda i, c: (0, c)),
            pl.BlockSpec((NB_ROWS, 1), lambda i, c: (i, 0)),
        ],
        out_specs=pl.BlockSpec((NB_ROWS, hd), lambda i, c: (c * NBLKS + i, 0)),
        out_shape=jax.ShapeDtypeStruct((NC * N_NODES, hd), jnp.float32),
    )(x, w, dinv_col)


def _mid_body(acc_lo, acc_hi, hp_lo, hp_hi, dinv_ref, b_ref, w_ref, out_ref):
    x = jnp.concatenate([acc_lo[...] + hp_lo[...], acc_hi[...] + hp_hi[...]],
                        axis=1)
    x = jnp.maximum(dinv_ref[...] * x + b_ref[...], 0.0)
    h = jnp.dot(x, w_ref[...], preferred_element_type=jnp.float32)
    out_ref[...] = dinv_ref[...] * h


def _layer_mid(acc, hp, dinv_col, b_row, w, hd_in, hd_out):
    half = pl.BlockSpec((NB_ROWS, hd_in), lambda i, c: (i, 0))
    half_hi = pl.BlockSpec((NB_ROWS, hd_in), lambda i, c: (NBLKS + i, 0))
    return pl.pallas_call(
        _mid_body,
        grid=(NBLKS, NC),
        in_specs=[
            half, half_hi, half, half_hi,
            pl.BlockSpec((NB_ROWS, 1), lambda i, c: (i, 0)),
            pl.BlockSpec((1, 2 * hd_in), lambda i, c: (0, 0)),
            pl.BlockSpec((2 * hd_in, hd_out), lambda i, c: (0, c)),
        ],
        out_specs=pl.BlockSpec((NB_ROWS, hd_out),
                               lambda i, c: (c * NBLKS + i, 0)),
        out_shape=jax.ShapeDtypeStruct((NC * N_NODES, hd_out), jnp.float32),
    )(acc, acc, hp, hp, dinv_col, b_row, w)


def _fin_body(acc_lo, acc_hi, hp_lo, hp_hi, dinv_ref, b_ref, out_ref):
    x = jnp.concatenate([acc_lo[...] + hp_lo[...], acc_hi[...] + hp_hi[...]],
                        axis=1)
    out_ref[...] = dinv_ref[...] * x + b_ref[...]


def _layer_fin(acc, hp, dinv_col, b_row, hd_in):
    half = pl.BlockSpec((NB_ROWS, hd_in), lambda i: (i, 0))
    half_hi = pl.BlockSpec((NB_ROWS, hd_in), lambda i: (NBLKS + i, 0))
    return pl.pallas_call(
        _fin_body,
        grid=(NBLKS,),
        in_specs=[
            half, half_hi, half, half_hi,
            pl.BlockSpec((NB_ROWS, 1), lambda i: (i, 0)),
            pl.BlockSpec((1, 2 * hd_in), lambda i: (0, 0)),
        ],
        out_specs=pl.BlockSpec((NB_ROWS, 2 * hd_in), lambda i: (i, 0)),
        out_shape=jax.ShapeDtypeStruct((N_NODES, 2 * hd_in), jnp.float32),
    )(acc, acc, hp, hp, dinv_col, b_row)


def kernel(edge_index, node_emb, W1, b1, W2, b2, W3, b3):
    src = edge_index[0].astype(jnp.int32)
    dst = edge_index[1].astype(jnp.int32)

    dinv_pad = _make_deg_kernel()(dst)
    dinv_col = dinv_pad[:N_NODES].reshape(N_NODES, 1)

    agg128 = _make_agg_kernel(128)
    agg64 = _make_agg_kernel(64)

    hp1 = _layer1(node_emb, W1, dinv_col, 128)
    acc1 = agg128(hp1, src, dst)
    hp2 = _layer_mid(acc1, hp1, dinv_col, b1.reshape(1, -1), W2, 128, 128)
    acc2 = agg128(hp2, src, dst)
    hp3 = _layer_mid(acc2, hp2, dinv_col, b2.reshape(1, -1), W3, 128, 64)
    acc3 = agg64(hp3, src, dst)
    return _layer_fin(acc3, hp3, dinv_col, b3.reshape(1, -1), 64)


# speedup vs baseline: 9.9490x; 9.9490x over previous
import functools

import jax
import jax.numpy as jnp
from jax import lax
from jax.experimental import pallas as pl
from jax.experimental.pallas import tpu as pltpu
from jax.experimental.pallas import tpu_sc as plsc

N_NODES = 10000
N_EDGES = 320000
NC, NS, L = 2, 16, 16

E_PER_SUB = N_EDGES // NS
EPB = 128
N_FULL_BLK = E_PER_SUB // EPB
EPB_TAIL = E_PER_SUB - N_FULL_BLK * EPB

NACC = 10240
RPS = NACC // NS

NPAD_DEG = 16384
DEG_SPAN = NPAD_DEG // NS

_MESH = plsc.VectorSubcoreMesh(
    core_axis_name="c", subcore_axis_name="s", num_cores=NC, num_subcores=NS)


def _deg_body(dst_hbm, cnt_hbm, ones_b, dbuf, dbuf_t, redbuf, obuf, cnt, sem):
    c = lax.axis_index("c")
    s = lax.axis_index("s")

    @pl.when(c == 0)
    def _core0():
        def fill_body(i, carry):
            ones_b[pl.ds(i * L, L)] = jnp.ones((L,), jnp.float32)
            obuf[pl.ds(i * L, L)] = jnp.zeros((L,), jnp.float32)
            return carry
        lax.fori_loop(0, DEG_SPAN // L, fill_body, 0)

        pltpu.sync_copy(obuf.at[pl.ds(0, DEG_SPAN)],
                        cnt.at[pl.ds(s * DEG_SPAN, DEG_SPAN)])
        plsc.subcore_barrier()

        e0 = s * E_PER_SUB

        def blk_body(i, carry):
            pltpu.sync_copy(dst_hbm.at[pl.ds(e0 + i * EPB, EPB)], dbuf)
            pltpu.sync_copy(ones_b.at[pl.ds(0, EPB)], cnt.at[dbuf], add=True)
            return carry
        lax.fori_loop(0, N_FULL_BLK, blk_body, 0)

        t_off = e0 + N_FULL_BLK * EPB
        pltpu.sync_copy(dst_hbm.at[pl.ds(t_off, EPB_TAIL)], dbuf_t)
        pltpu.sync_copy(ones_b.at[pl.ds(0, EPB_TAIL)], cnt.at[dbuf_t],
                        add=True)
        plsc.subcore_barrier()

        g0 = s * DEG_SPAN
        pltpu.sync_copy(cnt.at[pl.ds(g0, DEG_SPAN)], redbuf)
        pltpu.sync_copy(redbuf, cnt_hbm.at[pl.ds(g0, DEG_SPAN)])


def _make_deg_kernel():
    return pl.kernel(
        _deg_body,
        out_type=jax.ShapeDtypeStruct((NPAD_DEG,), jnp.float32),
        mesh=_MESH,
        scratch_types=[
            pltpu.VMEM((DEG_SPAN,), jnp.float32),
            pltpu.VMEM((EPB,), jnp.int32),
            pltpu.VMEM((EPB_TAIL,), jnp.int32),
            pltpu.VMEM((DEG_SPAN,), jnp.float32),
            pltpu.VMEM((DEG_SPAN,), jnp.float32),
            pltpu.VMEM_SHARED((NPAD_DEG,), jnp.float32),
            pltpu.SemaphoreType.DMA,
        ],
    )


def _agg_body(hd, hp_hbm, src_hbm, dst_hbm, out_hbm,
              sbuf, dbuf, sbuf_t, dbuf_t, rows, acc, sem):
    c = lax.axis_index("c")
    s = lax.axis_index("s")
    c_off = c * N_NODES

    def zfill(i, carry):
        for j in range(hd // L):
            rows[i, pl.ds(j * L, L)] = jnp.zeros((L,), jnp.float32)
        return carry
    lax.fori_loop(0, EPB, zfill, 0)

    r0 = s * RPS
    for k in range(RPS // EPB):
        pltpu.sync_copy(rows, acc.at[pl.ds(r0 + k * EPB, EPB)])
    plsc.subcore_barrier()

    e0 = s * E_PER_SUB

    def blk_body(i, carry):
        off = e0 + i * EPB
        pltpu.sync_copy(src_hbm.at[pl.ds(off, EPB)], sbuf)
        pltpu.sync_copy(dst_hbm.at[pl.ds(off, EPB)], dbuf)
        for j in range(EPB // L):
            sbuf[pl.ds(j * L, L)] = sbuf[pl.ds(j * L, L)] + c_off
        pltpu.async_copy(hp_hbm.at[sbuf], rows, sem).wait()
        pltpu.sync_copy(rows, acc.at[dbuf], add=True)
        return carry
    lax.fori_loop(0, N_FULL_BLK, blk_body, 0)

    t_off = e0 + N_FULL_BLK * EPB
    pltpu.sync_copy(src_hbm.at[pl.ds(t_off, EPB_TAIL)], sbuf_t)
    pltpu.sync_copy(dst_hbm.at[pl.ds(t_off, EPB_TAIL)], dbuf_t)
    for j in range(EPB_TAIL // L):
        sbuf_t[pl.ds(j * L, L)] = sbuf_t[pl.ds(j * L, L)] + c_off
    pltpu.async_copy(hp_hbm.at[sbuf_t], rows.at[pl.ds(0, EPB_TAIL)], sem).wait()
    pltpu.sync_copy(rows.at[pl.ds(0, EPB_TAIL)], acc.at[dbuf_t], add=True)

    plsc.subcore_barrier()

    for k in range(RPS // EPB):
        rr = r0 + k * EPB
        pltpu.sync_copy(acc.at[pl.ds(rr, EPB)], rows)
        pltpu.sync_copy(rows, out_hbm.at[pl.ds(c * NACC + rr, EPB)])


E_PER_CORE = N_EDGES // NC
E_PER_SUB_F = E_PER_CORE // NS
N_FULL_BLK_F = E_PER_SUB_F // EPB
EPB_TAIL_F = E_PER_SUB_F - N_FULL_BLK_F * EPB


def _agg_full_body(hp_hbm, src_hbm, dst_hbm, out_hbm,
                   sbuf, dbuf, sbuf_t, dbuf_t, rows, acc, sem):
    c = lax.axis_index("c")
    s = lax.axis_index("s")
    hd = 128

    def zfill(i, carry):
        for j in range(hd // L):
            rows[i, pl.ds(j * L, L)] = jnp.zeros((L,), jnp.float32)
        return carry
    lax.fori_loop(0, EPB, zfill, 0)

    r0 = s * RPS
    for k in range(RPS // EPB):
        pltpu.sync_copy(rows, acc.at[pl.ds(r0 + k * EPB, EPB)])
    plsc.subcore_barrier()

    e0 = c * E_PER_CORE + s * E_PER_SUB_F

    def blk_body(i, carry):
        off = e0 + i * EPB
        pltpu.sync_copy(src_hbm.at[pl.ds(off, EPB)], sbuf)
        pltpu.sync_copy(dst_hbm.at[pl.ds(off, EPB)], dbuf)
        pltpu.async_copy(hp_hbm.at[sbuf], rows, sem).wait()
        pltpu.sync_copy(rows, acc.at[dbuf], add=True)
        return carry
    lax.fori_loop(0, N_FULL_BLK_F, blk_body, 0)

    t_off = e0 + N_FULL_BLK_F * EPB
    pltpu.sync_copy(src_hbm.at[pl.ds(t_off, EPB_TAIL_F)], sbuf_t)
    pltpu.sync_copy(dst_hbm.at[pl.ds(t_off, EPB_TAIL_F)], dbuf_t)
    pltpu.async_copy(hp_hbm.at[sbuf_t], rows.at[pl.ds(0, EPB_TAIL_F)],
                     sem).wait()
    pltpu.sync_copy(rows.at[pl.ds(0, EPB_TAIL_F)], acc.at[dbuf_t], add=True)

    plsc.subcore_barrier()

    for k in range(RPS // EPB):
        rr = r0 + k * EPB
        pltpu.sync_copy(acc.at[pl.ds(rr, EPB)], rows)
        pltpu.sync_copy(rows, out_hbm.at[pl.ds(c * NACC + rr, EPB)])


def _make_agg_full_kernel():
    return pl.kernel(
        _agg_full_body,
        out_type=jax.ShapeDtypeStruct((NC * NACC, 128), jnp.float32),
        mesh=_MESH,
        scratch_types=[
            pltpu.VMEM((EPB,), jnp.int32),
            pltpu.VMEM((EPB,), jnp.int32),
            pltpu.VMEM((EPB_TAIL_F,), jnp.int32),
            pltpu.VMEM((EPB_TAIL_F,), jnp.int32),
            pltpu.VMEM((EPB, 128), jnp.float32),
            pltpu.VMEM_SHARED((NACC, 128), jnp.float32),
            pltpu.SemaphoreType.DMA,
        ],
    )


def _make_agg_kernel(hd):
    return pl.kernel(
        functools.partial(_agg_body, hd),
        out_type=jax.ShapeDtypeStruct((NC * NACC, hd), jnp.float32),
        mesh=_MESH,
        scratch_types=[
            pltpu.VMEM((EPB,), jnp.int32),
            pltpu.VMEM((EPB,), jnp.int32),
            pltpu.VMEM((EPB_TAIL,), jnp.int32),
            pltpu.VMEM((EPB_TAIL,), jnp.int32),
            pltpu.VMEM((EPB, hd), jnp.float32),
            pltpu.VMEM_SHARED((NACC, hd), jnp.float32),
            pltpu.SemaphoreType.DMA,
        ],
    )


NB_ROWS = 400
NBLKS = N_NODES // NB_ROWS


def _dinv_body(cnt_ref, out_ref):
    out_ref[...] = lax.rsqrt(cnt_ref[...] + 1.0)


def _dinv_tc(cnt_pad):
    cnt2d = cnt_pad.reshape(NPAD_DEG // 128, 128)
    return pl.pallas_call(
        _dinv_body,
        out_shape=jax.ShapeDtypeStruct((NPAD_DEG // 128, 128), jnp.float32),
    )(cnt2d)


def _l1_body(x_ref, w_ref, dinv_ref, out_ref):
    h = jnp.dot(x_ref[...], w_ref[0], preferred_element_type=jnp.float32)
    out_ref[...] = dinv_ref[...] * h


def _layer1(x, w_split, dinv_col, hd):
    in_dim = x.shape[1]
    return pl.pallas_call(
        _l1_body,
        grid=(NBLKS, NC),
        in_specs=[
            pl.BlockSpec((NB_ROWS, in_dim), lambda i, c: (i, 0)),
            pl.BlockSpec((1, in_dim, hd), lambda i, c: (c, 0, 0)),
            pl.BlockSpec((NB_ROWS, 1), lambda i, c: (i, 0)),
        ],
        out_specs=pl.BlockSpec((NB_ROWS, hd), lambda i, c: (c * NBLKS + i, 0)),
        out_shape=jax.ShapeDtypeStruct((NC * N_NODES, hd), jnp.float32),
    )(x, w_split, dinv_col)


def _mid_body(acc_lo, acc_hi, hp_lo, hp_hi, dinv_ref, b_ref, w_ref, out_ref):
    x = jnp.concatenate([acc_lo[...] + hp_lo[...], acc_hi[...] + hp_hi[...]],
                        axis=1)
    x = jnp.maximum(dinv_ref[...] * x + b_ref[...], 0.0)
    h = jnp.dot(x, w_ref[0], preferred_element_type=jnp.float32)
    out_ref[...] = dinv_ref[...] * h


def _layer_mid(acc_lo, acc_hi, hp, dinv_col, b_row, w_split, hd_in, hd_out):
    lo = pl.BlockSpec((NB_ROWS, hd_in), lambda i, c: (i, 0))
    hp_hi = pl.BlockSpec((NB_ROWS, hd_in), lambda i, c: (NBLKS + i, 0))
    return pl.pallas_call(
        _mid_body,
        grid=(NBLKS, NC),
        in_specs=[
            lo, lo, lo, hp_hi,
            pl.BlockSpec((NB_ROWS, 1), lambda i, c: (i, 0)),
            pl.BlockSpec((1, 2 * hd_in), lambda i, c: (0, 0)),
            pl.BlockSpec((1, 2 * hd_in, hd_out), lambda i, c: (c, 0, 0)),
        ],
        out_specs=pl.BlockSpec((NB_ROWS, hd_out),
                               lambda i, c: (c * NBLKS + i, 0)),
        out_shape=jax.ShapeDtypeStruct((NC * N_NODES, hd_out), jnp.float32),
    )(acc_lo, acc_hi, hp, hp, dinv_col, b_row, w_split)


def _mid1_body(acc_lo, acc_hi, hp_lo, hp_hi, dinv_ref, b_ref, w_ref, out_ref):
    x = jnp.concatenate([acc_lo[...] + hp_lo[...], acc_hi[...] + hp_hi[...]],
                        axis=1)
    x = jnp.maximum(dinv_ref[...] * x + b_ref[...], 0.0)
    h = jnp.dot(x, w_ref[...], preferred_element_type=jnp.float32)
    out_ref[...] = dinv_ref[...] * h


def _layer_mid_full(acc_lo, acc_hi, hp, dinv_col, b_row, w, hd_in, hd_out):
    lo = pl.BlockSpec((NB_ROWS, hd_in), lambda i: (i, 0))
    hp_hi = pl.BlockSpec((NB_ROWS, hd_in), lambda i: (NBLKS + i, 0))
    return pl.pallas_call(
        _mid1_body,
        grid=(NBLKS,),
        in_specs=[
            lo, lo, lo, hp_hi,
            pl.BlockSpec((NB_ROWS, 1), lambda i: (i, 0)),
            pl.BlockSpec((1, 2 * hd_in), lambda i: (0, 0)),
            pl.BlockSpec((2 * hd_in, hd_out), lambda i: (0, 0)),
        ],
        out_specs=pl.BlockSpec((NB_ROWS, hd_out), lambda i: (i, 0)),
        out_shape=jax.ShapeDtypeStruct((N_NODES, hd_out), jnp.float32),
    )(acc_lo, acc_hi, hp, hp, dinv_col, b_row, w)


def _fin_body(acc_a, acc_b, hp_ref, dinv_ref, b_ref, out_ref):
    x = acc_a[...] + acc_b[...] + hp_ref[...]
    out_ref[...] = dinv_ref[...] * x + b_ref[...]


def _layer_fin(acc_a, acc_b, hp, dinv_col, b_row, hd):
    full = pl.BlockSpec((NB_ROWS, hd), lambda i: (i, 0))
    return pl.pallas_call(
        _fin_body,
        grid=(NBLKS,),
        in_specs=[
            full, full, full,
            pl.BlockSpec((NB_ROWS, 1), lambda i: (i, 0)),
            pl.BlockSpec((1, hd), lambda i: (0, 0)),
        ],
        out_specs=full,
        out_shape=jax.ShapeDtypeStruct((N_NODES, hd), jnp.float32),
    )(acc_a, acc_b, hp, dinv_col, b_row)


def kernel(edge_index, node_emb, W1, b1, W2, b2, W3, b3):
    src = edge_index[0].astype(jnp.int32)
    dst = edge_index[1].astype(jnp.int32)

    cnt_pad = _make_deg_kernel()(dst)
    dinv_col = _dinv_tc(cnt_pad).reshape(-1)[:N_NODES].reshape(N_NODES, 1)

    agg128 = _make_agg_kernel(128)
    agg_full = _make_agg_full_kernel()

    def split_cols(w, hd):
        return jnp.stack([w[:, :hd], w[:, hd:]])

    def halves(acc_pad):
        return acc_pad[:N_NODES], acc_pad[NACC:NACC + N_NODES]

    hp1 = _layer1(node_emb, split_cols(W1, 128), dinv_col, 128)
    a1_lo, a1_hi = halves(agg128(hp1, src, dst))
    hp2 = _layer_mid(a1_lo, a1_hi, hp1, dinv_col, b1.reshape(1, -1),
                     split_cols(W2, 128), 128, 128)
    a2_lo, a2_hi = halves(agg128(hp2, src, dst))
    hp3 = _layer_mid_full(a2_lo, a2_hi, hp2, dinv_col, b2.reshape(1, -1),
                          W3, 128, 128)
    a3_a, a3_b = halves(agg_full(hp3, src, dst))
    return _layer_fin(a3_a, a3_b, hp3, dinv_col, b3.reshape(1, -1), 128)


# speedup vs baseline: 13.8798x; 1.3951x over previous
import functools

import jax
import jax.numpy as jnp
from jax import lax
from jax.experimental import pallas as pl
from jax.experimental.pallas import tpu as pltpu
from jax.experimental.pallas import tpu_sc as plsc

N_NODES = 10000
N_EDGES = 320000
NC, NS, L = 2, 16, 16

EPB = 125
NBLK = N_EDGES // EPB
BPS = NBLK // NS
BPW = NBLK // (NC * NS)

NACC = 10240
RPS = NACC // NS
ZCH = 64
CB = 16

NPAD_DEG = 16384
DEG_SPAN = NPAD_DEG // NS

_MESH = plsc.VectorSubcoreMesh(
    core_axis_name="c", subcore_axis_name="s", num_cores=NC, num_subcores=NS)


def _deg_body(dst_hbm, cnt_hbm, ones_b, didx, redbuf, cnt, sem):
    c = lax.axis_index("c")
    s = lax.axis_index("s")

    def fill_body(i, carry):
        ones_b[pl.ds(i * L, L)] = jnp.ones((L,), jnp.float32)
        redbuf[pl.ds(i * L, L)] = jnp.zeros((L,), jnp.float32)
        return carry
    lax.fori_loop(0, DEG_SPAN // L, fill_body, 0)

    pltpu.sync_copy(redbuf, cnt.at[pl.ds(s * DEG_SPAN, DEG_SPAN)])
    b0 = (c * NS + s) * BPW
    pltpu.sync_copy(dst_hbm.at[pl.ds(b0, BPW)], didx)
    plsc.subcore_barrier()

    src = ones_b.at[pl.ds(0, EPB)]

    def blk_body(i, carry):
        for k in range(8):
            pltpu.async_copy(src, cnt.at[didx.at[i * 8 + k]], sem, add=True)
        for k in range(8):
            pltpu.make_async_copy(src, cnt.at[didx.at[0]], sem).wait()
        return carry
    lax.fori_loop(0, BPW // 8, blk_body, 0)
    plsc.subcore_barrier()

    g0 = s * DEG_SPAN
    pltpu.sync_copy(cnt.at[pl.ds(g0, DEG_SPAN)], redbuf)
    pltpu.sync_copy(redbuf, cnt_hbm.at[pl.ds(c * NPAD_DEG + g0, DEG_SPAN)])


def _make_deg_kernel():
    return pl.kernel(
        _deg_body,
        out_type=jax.ShapeDtypeStruct((NC * NPAD_DEG,), jnp.float32),
        mesh=_MESH,
        scratch_types=[
            pltpu.VMEM((DEG_SPAN,), jnp.float32),
            pltpu.VMEM((BPW, EPB), jnp.int32),
            pltpu.VMEM((DEG_SPAN,), jnp.float32),
            pltpu.VMEM_SHARED((NPAD_DEG,), jnp.float32),
            pltpu.SemaphoreType.DMA,
        ],
    )


def _agg_loop(hp_hbm, acc, src_blk_hbm, dst_blk_hbm, b0, nblk,
              sidx, didx, rows_a, rows_b, gsa, gsb, ssa, ssb):
    def chunk_body(q, carry):
        cb0 = b0 + q * CB
        pltpu.sync_copy(src_blk_hbm.at[pl.ds(cb0, CB)], sidx)
        pltpu.sync_copy(dst_blk_hbm.at[pl.ds(cb0, CB)], didx)
        pltpu.async_copy(hp_hbm.at[sidx.at[0]], rows_a, gsa)

        def pair_body(i, carry2):
            @pl.when(i > 0)
            def _():
                pltpu.make_async_copy(rows_b, acc.at[didx.at[0]], ssb).wait()
            pltpu.async_copy(hp_hbm.at[sidx.at[2 * i + 1]], rows_b, gsb)
            pltpu.make_async_copy(hp_hbm.at[sidx.at[0]], rows_a, gsa).wait()
            pltpu.async_copy(rows_a, acc.at[didx.at[2 * i]], ssa, add=True)
            pltpu.make_async_copy(rows_a, acc.at[didx.at[0]], ssa).wait()

            @pl.when(i < CB // 2 - 1)
            def _():
                pltpu.async_copy(hp_hbm.at[sidx.at[2 * i + 2]], rows_a, gsa)
            pltpu.make_async_copy(hp_hbm.at[sidx.at[0]], rows_b, gsb).wait()
            pltpu.async_copy(rows_b, acc.at[didx.at[2 * i + 1]], ssb, add=True)
            return carry2
        lax.fori_loop(0, CB // 2, pair_body, 0)
        pltpu.make_async_copy(rows_b, acc.at[didx.at[0]], ssb).wait()
        return carry
    lax.fori_loop(0, nblk // CB, chunk_body, 0)


def _zero_acc(zbuf, acc, s):
    def zfill(i, carry):
        for j in range(128 // L):
            zbuf[i, pl.ds(j * L, L)] = jnp.zeros((L,), jnp.float32)
        return carry
    lax.fori_loop(0, ZCH, zfill, 0)
    r0 = s * RPS
    for k in range(RPS // ZCH):
        rr = r0 + k * ZCH

        @pl.when(rr < N_NODES)
        def _():
            pltpu.sync_copy(zbuf, acc.at[pl.ds(rr, ZCH)])


def _write_acc(zbuf, acc, out_hbm, c, s):
    r0 = s * RPS
    for k in range(RPS // ZCH):
        rr = r0 + k * ZCH

        @pl.when(rr < N_NODES)
        def _():
            pltpu.sync_copy(acc.at[pl.ds(rr, ZCH)], zbuf)
            pltpu.sync_copy(zbuf, out_hbm.at[pl.ds(c * NACC + rr, ZCH)])


def _agg_split_body(hp_hbm, src_hbm, srchi_hbm, dst_hbm, out_hbm,
                    sidx, didx, rows_a, rows_b, zbuf, acc,
                    gsa, gsb, ssa, ssb):
    c = lax.axis_index("c")
    s = lax.axis_index("s")
    _zero_acc(zbuf, acc, s)
    plsc.subcore_barrier()

    b0 = s * BPS

    @pl.when(c == 0)
    def _():
        _agg_loop(hp_hbm, acc, src_hbm, dst_hbm, b0, BPS,
                  sidx, didx, rows_a, rows_b, gsa, gsb, ssa, ssb)

    @pl.when(c == 1)
    def _():
        _agg_loop(hp_hbm, acc, srchi_hbm, dst_hbm, b0, BPS,
                  sidx, didx, rows_a, rows_b, gsa, gsb, ssa, ssb)
    plsc.subcore_barrier()
    _write_acc(zbuf, acc, out_hbm, c, s)


def _agg_edges_body(hp_hbm, src_hbm, dst_hbm, out_hbm,
                    sidx, didx, rows_a, rows_b, zbuf, acc,
                    gsa, gsb, ssa, ssb):
    c = lax.axis_index("c")
    s = lax.axis_index("s")
    _zero_acc(zbuf, acc, s)
    plsc.subcore_barrier()

    b0 = (c * NS + s) * BPW
    _agg_loop(hp_hbm, acc, src_hbm, dst_hbm, b0, BPW,
              sidx, didx, rows_a, rows_b, gsa, gsb, ssa, ssb)
    plsc.subcore_barrier()
    _write_acc(zbuf, acc, out_hbm, c, s)


def _agg_scratch():
    return [
        pltpu.VMEM((CB, EPB), jnp.int32),
        pltpu.VMEM((CB, EPB), jnp.int32),
        pltpu.VMEM((EPB, 128), jnp.float32),
        pltpu.VMEM((EPB, 128), jnp.float32),
        pltpu.VMEM((ZCH, 128), jnp.float32),
        pltpu.VMEM_SHARED((NACC, 128), jnp.float32),
        pltpu.SemaphoreType.DMA,
        pltpu.SemaphoreType.DMA,
        pltpu.SemaphoreType.DMA,
        pltpu.SemaphoreType.DMA,
    ]


def _make_agg_split_kernel():
    return pl.kernel(
        _agg_split_body,
        out_type=jax.ShapeDtypeStruct((NC * NACC, 128), jnp.float32),
        mesh=_MESH,
        scratch_types=_agg_scratch(),
    )


def _make_agg_edges_kernel():
    return pl.kernel(
        _agg_edges_body,
        out_type=jax.ShapeDtypeStruct((NC * NACC, 128), jnp.float32),
        mesh=_MESH,
        scratch_types=_agg_scratch(),
    )


NB_ROWS = 80
NBLKS = N_NODES // NB_ROWS
HI_OFF = NACC // NB_ROWS


def _dinv_body(cnt_ref, out_ref):
    out_ref[...] = lax.rsqrt(cnt_ref[0] + cnt_ref[1] + 1.0)


def _dinv_tc(cnt_pad):
    cnt3d = cnt_pad.reshape(NC, NPAD_DEG // 128, 128)
    return pl.pallas_call(
        _dinv_body,
        out_shape=jax.ShapeDtypeStruct((NPAD_DEG // 128, 128), jnp.float32),
    )(cnt3d)


def _l1_body(x_ref, w_ref, dinv_ref, out_ref):
    h = jnp.dot(x_ref[...], w_ref[0], preferred_element_type=jnp.float32)
    out_ref[...] = dinv_ref[...] * h


def _layer1(x, w_split, dinv_col, hd):
    in_dim = x.shape[1]
    return pl.pallas_call(
        _l1_body,
        grid=(NC, NBLKS),
        in_specs=[
            pl.BlockSpec((NB_ROWS, in_dim), lambda c, i: (i, 0)),
            pl.BlockSpec((1, in_dim, hd), lambda c, i: (c, 0, 0)),
            pl.BlockSpec((NB_ROWS, 1), lambda c, i: (i, 0)),
        ],
        out_specs=pl.BlockSpec((NB_ROWS, hd), lambda c, i: (c * NBLKS + i, 0)),
        out_shape=jax.ShapeDtypeStruct((NC * N_NODES, hd), jnp.float32),
    )(x, w_split, dinv_col)


def _mid_body(acc_lo, acc_hi, hp_lo, hp_hi, dinv_ref, b_ref, w_ref, out_ref):
    x = jnp.concatenate([acc_lo[...] + hp_lo[...], acc_hi[...] + hp_hi[...]],
                        axis=1)
    x = jnp.maximum(dinv_ref[...] * x + b_ref[...], 0.0)
    h = jnp.dot(x, w_ref[0], preferred_element_type=jnp.float32)
    out_ref[...] = dinv_ref[...] * h


def _layer_mid(acc_pad, hp, dinv_col, b_row, w_split, hd_in, hd_out):
    acc_lo = pl.BlockSpec((NB_ROWS, hd_in), lambda c, i: (i, 0))
    acc_hi = pl.BlockSpec((NB_ROWS, hd_in), lambda c, i: (HI_OFF + i, 0))
    hp_lo = pl.BlockSpec((NB_ROWS, hd_in), lambda c, i: (i, 0))
    hp_hi = pl.BlockSpec((NB_ROWS, hd_in), lambda c, i: (NBLKS + i, 0))
    return pl.pallas_call(
        _mid_body,
        grid=(NC, NBLKS),
        in_specs=[
            acc_lo, acc_hi, hp_lo, hp_hi,
            pl.BlockSpec((NB_ROWS, 1), lambda c, i: (i, 0)),
            pl.BlockSpec((1, 2 * hd_in), lambda c, i: (0, 0)),
            pl.BlockSpec((1, 2 * hd_in, hd_out), lambda c, i: (c, 0, 0)),
        ],
        out_specs=pl.BlockSpec((NB_ROWS, hd_out),
                               lambda c, i: (c * NBLKS + i, 0)),
        out_shape=jax.ShapeDtypeStruct((NC * N_NODES, hd_out), jnp.float32),
    )(acc_pad, acc_pad, hp, hp, dinv_col, b_row, w_split)


def _mid1_body(acc_lo, acc_hi, hp_lo, hp_hi, dinv_ref, b_ref, w_ref, out_ref):
    x = jnp.concatenate([acc_lo[...] + hp_lo[...], acc_hi[...] + hp_hi[...]],
                        axis=1)
    x = jnp.maximum(dinv_ref[...] * x + b_ref[...], 0.0)
    h = jnp.dot(x, w_ref[...], preferred_element_type=jnp.float32)
    out_ref[...] = dinv_ref[...] * h


def _layer_mid_full(acc_pad, hp, dinv_col, b_row, w, hd_in, hd_out):
    acc_lo = pl.BlockSpec((NB_ROWS, hd_in), lambda i: (i, 0))
    acc_hi = pl.BlockSpec((NB_ROWS, hd_in), lambda i: (HI_OFF + i, 0))
    hp_lo = pl.BlockSpec((NB_ROWS, hd_in), lambda i: (i, 0))
    hp_hi = pl.BlockSpec((NB_ROWS, hd_in), lambda i: (NBLKS + i, 0))
    return pl.pallas_call(
        _mid1_body,
        grid=(NBLKS,),
        in_specs=[
            acc_lo, acc_hi, hp_lo, hp_hi,
            pl.BlockSpec((NB_ROWS, 1), lambda i: (i, 0)),
            pl.BlockSpec((1, 2 * hd_in), lambda i: (0, 0)),
            pl.BlockSpec((2 * hd_in, hd_out), lambda i: (0, 0)),
        ],
        out_specs=pl.BlockSpec((NB_ROWS, hd_out), lambda i: (i, 0)),
        out_shape=jax.ShapeDtypeStruct((N_NODES, hd_out), jnp.float32),
    )(acc_pad, acc_pad, hp, hp, dinv_col, b_row, w)


def _fin_body(acc_a, acc_b, hp_ref, dinv_ref, b_ref, out_ref):
    x = acc_a[...] + acc_b[...] + hp_ref[...]
    out_ref[...] = dinv_ref[...] * x + b_ref[...]


def _layer_fin(acc_pad, hp, dinv_col, b_row, hd):
    full = pl.BlockSpec((NB_ROWS, hd), lambda i: (i, 0))
    acc_hi = pl.BlockSpec((NB_ROWS, hd), lambda i: (HI_OFF + i, 0))
    return pl.pallas_call(
        _fin_body,
        grid=(NBLKS,),
        in_specs=[
            full, acc_hi, full,
            pl.BlockSpec((NB_ROWS, 1), lambda i: (i, 0)),
            pl.BlockSpec((1, hd), lambda i: (0, 0)),
        ],
        out_specs=full,
        out_shape=jax.ShapeDtypeStruct((N_NODES, hd), jnp.float32),
    )(acc_pad, acc_pad, hp, dinv_col, b_row)


def kernel(edge_index, node_emb, W1, b1, W2, b2, W3, b3):
    src = edge_index[0].astype(jnp.int32)
    dst = edge_index[1].astype(jnp.int32)
    src2d = src.reshape(NBLK, EPB)
    srchi2d = (src + N_NODES).reshape(NBLK, EPB)
    dst2d = dst.reshape(NBLK, EPB)

    cnt_pad = _make_deg_kernel()(dst2d)
    dinv_col = _dinv_tc(cnt_pad).reshape(-1)[:N_NODES].reshape(N_NODES, 1)

    agg_split = _make_agg_split_kernel()
    agg_edges = _make_agg_edges_kernel()

    def split_cols(w, hd):
        return jnp.stack([w[:, :hd], w[:, hd:]])

    hp1 = _layer1(node_emb, split_cols(W1, 128), dinv_col, 128)
    a1 = agg_split(hp1, src2d, srchi2d, dst2d)
    hp2 = _layer_mid(a1, hp1, dinv_col, b1.reshape(1, -1),
                     split_cols(W2, 128), 128, 128)
    a2 = agg_split(hp2, src2d, srchi2d, dst2d)
    hp3 = _layer_mid_full(a2, hp2, dinv_col, b2.reshape(1, -1),
                          W3, 128, 128)
    a3 = agg_edges(hp3, src2d, dst2d)
    return _layer_fin(a3, hp3, dinv_col, b3.reshape(1, -1), 128)


# speedup vs baseline: 20.0360x; 1.4435x over previous
import functools

import jax
import jax.numpy as jnp
from jax import lax
from jax.experimental import pallas as pl
from jax.experimental.pallas import tpu as pltpu
from jax.experimental.pallas import tpu_sc as plsc

N_NODES = 10000
N_EDGES = 320000
NC, NS, L = 2, 16, 16

EPB = 125
NBLK = N_EDGES // EPB
BPS = NBLK // NS
BPW = NBLK // (NC * NS)

WCH = 80
NWCH = N_NODES // WCH
CB = 16

NPAD_DEG = 16384
DEG_SPAN = NPAD_DEG // NS

_MESH = plsc.VectorSubcoreMesh(
    core_axis_name="c", subcore_axis_name="s", num_cores=NC, num_subcores=NS)


def _deg_body(dst_hbm, cnt_hbm, ones_b, didx, redbuf, cnt, sem):
    c = lax.axis_index("c")
    s = lax.axis_index("s")

    def fill_body(i, carry):
        ones_b[pl.ds(i * L, L)] = jnp.ones((L,), jnp.float32)
        redbuf[pl.ds(i * L, L)] = jnp.zeros((L,), jnp.float32)
        return carry
    lax.fori_loop(0, DEG_SPAN // L, fill_body, 0)

    pltpu.sync_copy(redbuf, cnt.at[pl.ds(s * DEG_SPAN, DEG_SPAN)])
    b0 = (c * NS + s) * BPW
    pltpu.sync_copy(dst_hbm.at[pl.ds(b0, BPW)], didx)
    plsc.subcore_barrier()

    src = ones_b.at[pl.ds(0, EPB)]

    def blk_body(i, carry):
        for k in range(8):
            pltpu.async_copy(src, cnt.at[didx.at[i * 8 + k]], sem, add=True)
        for k in range(8):
            pltpu.make_async_copy(src, cnt.at[didx.at[0]], sem).wait()
        return carry
    lax.fori_loop(0, BPW // 8, blk_body, 0)
    plsc.subcore_barrier()

    g0 = s * DEG_SPAN
    pltpu.sync_copy(cnt.at[pl.ds(g0, DEG_SPAN)], redbuf)
    pltpu.sync_copy(redbuf, cnt_hbm.at[pl.ds(c * NPAD_DEG + g0, DEG_SPAN)])


def _make_deg_kernel():
    return pl.kernel(
        _deg_body,
        out_type=jax.ShapeDtypeStruct((NC * NPAD_DEG,), jnp.float32),
        mesh=_MESH,
        scratch_types=[
            pltpu.VMEM((DEG_SPAN,), jnp.float32),
            pltpu.VMEM((BPW, EPB), jnp.int32),
            pltpu.VMEM((DEG_SPAN,), jnp.float32),
            pltpu.VMEM_SHARED((NPAD_DEG,), jnp.float32),
            pltpu.SemaphoreType.DMA,
        ],
    )


def _agg_loop(hp_hbm, acc, src_blk_hbm, dst_blk_hbm, b0, nblk,
              sidx, didx, rows_a, rows_b, gsa, gsb, ssa, ssb):
    def chunk_body(q, carry):
        cb0 = b0 + q * CB
        pltpu.sync_copy(src_blk_hbm.at[pl.ds(cb0, CB)], sidx)
        pltpu.sync_copy(dst_blk_hbm.at[pl.ds(cb0, CB)], didx)
        pltpu.async_copy(hp_hbm.at[sidx.at[0]], rows_a, gsa)

        def pair_body(i, carry2):
            @pl.when(i > 0)
            def _():
                pltpu.make_async_copy(rows_b, acc.at[didx.at[0]], ssb).wait()
            pltpu.async_copy(hp_hbm.at[sidx.at[2 * i + 1]], rows_b, gsb)
            pltpu.make_async_copy(hp_hbm.at[sidx.at[0]], rows_a, gsa).wait()
            pltpu.async_copy(rows_a, acc.at[didx.at[2 * i]], ssa, add=True)
            pltpu.make_async_copy(rows_a, acc.at[didx.at[0]], ssa).wait()

            @pl.when(i < CB // 2 - 1)
            def _():
                pltpu.async_copy(hp_hbm.at[sidx.at[2 * i + 2]], rows_a, gsa)
            pltpu.make_async_copy(hp_hbm.at[sidx.at[0]], rows_b, gsb).wait()
            pltpu.async_copy(rows_b, acc.at[didx.at[2 * i + 1]], ssb, add=True)
            return carry2
        lax.fori_loop(0, CB // 2, pair_body, 0)
        pltpu.make_async_copy(rows_b, acc.at[didx.at[0]], ssb).wait()
        return carry
    lax.fori_loop(0, nblk // CB, chunk_body, 0)


def _zero_acc(zbuf, acc, s):
    def zfill(i, carry):
        for j in range(128 // L):
            zbuf[i, pl.ds(j * L, L)] = jnp.zeros((L,), jnp.float32)
        return carry
    lax.fori_loop(0, WCH, zfill, 0)
    for k in range((NWCH + NS - 1) // NS):
        j = k * NS + s

        @pl.when(j < NWCH)
        def _():
            pltpu.sync_copy(zbuf, acc.at[pl.ds(j * WCH, WCH)])


def _write_acc(zbuf, acc, out_hbm, c, s):
    for k in range((NWCH + NS - 1) // NS):
        j = k * NS + s

        @pl.when(j < NWCH)
        def _():
            pltpu.sync_copy(acc.at[pl.ds(j * WCH, WCH)], zbuf)
            pltpu.sync_copy(zbuf,
                            out_hbm.at[pl.ds(c * N_NODES + j * WCH, WCH)])


def _agg_split_body(hp_hbm, src_hbm, srchi_hbm, dst_hbm, out_hbm,
                    sidx, didx, rows_a, rows_b, zbuf, acc,
                    gsa, gsb, ssa, ssb):
    c = lax.axis_index("c")
    s = lax.axis_index("s")
    _zero_acc(zbuf, acc, s)
    plsc.subcore_barrier()

    b0 = s * BPS

    @pl.when(c == 0)
    def _():
        _agg_loop(hp_hbm, acc, src_hbm, dst_hbm, b0, BPS,
                  sidx, didx, rows_a, rows_b, gsa, gsb, ssa, ssb)

    @pl.when(c == 1)
    def _():
        _agg_loop(hp_hbm, acc, srchi_hbm, dst_hbm, b0, BPS,
                  sidx, didx, rows_a, rows_b, gsa, gsb, ssa, ssb)
    plsc.subcore_barrier()
    _write_acc(zbuf, acc, out_hbm, c, s)


def _agg_edges_body(hp_hbm, src_hbm, dst_hbm, out_hbm,
                    sidx, didx, rows_a, rows_b, zbuf, acc,
                    gsa, gsb, ssa, ssb):
    c = lax.axis_index("c")
    s = lax.axis_index("s")
    _zero_acc(zbuf, acc, s)
    plsc.subcore_barrier()

    b0 = (c * NS + s) * BPW
    _agg_loop(hp_hbm, acc, src_hbm, dst_hbm, b0, BPW,
              sidx, didx, rows_a, rows_b, gsa, gsb, ssa, ssb)
    plsc.subcore_barrier()
    _write_acc(zbuf, acc, out_hbm, c, s)


def _agg_scratch():
    return [
        pltpu.VMEM((CB, EPB), jnp.int32),
        pltpu.VMEM((CB, EPB), jnp.int32),
        pltpu.VMEM((EPB, 128), jnp.float32),
        pltpu.VMEM((EPB, 128), jnp.float32),
        pltpu.VMEM((WCH, 128), jnp.float32),
        pltpu.VMEM_SHARED((N_NODES, 128), jnp.float32),
        pltpu.SemaphoreType.DMA,
        pltpu.SemaphoreType.DMA,
        pltpu.SemaphoreType.DMA,
        pltpu.SemaphoreType.DMA,
    ]


def _make_agg_split_kernel():
    return pl.kernel(
        _agg_split_body,
        out_type=jax.ShapeDtypeStruct((NC * N_NODES, 128), jnp.float32),
        mesh=_MESH,
        scratch_types=_agg_scratch(),
    )


def _make_agg_edges_kernel():
    return pl.kernel(
        _agg_edges_body,
        out_type=jax.ShapeDtypeStruct((NC * N_NODES, 128), jnp.float32),
        mesh=_MESH,
        scratch_types=_agg_scratch(),
    )


NB_ROWS = 400
NBLKS = N_NODES // NB_ROWS
HI_OFF = NBLKS


def _dinv_body(cnt_ref, out_ref):
    out_ref[...] = lax.rsqrt(cnt_ref[0] + cnt_ref[1] + 1.0)


def _dinv_tc(cnt_pad):
    cnt3d = cnt_pad.reshape(NC, NPAD_DEG // 128, 128)
    return pl.pallas_call(
        _dinv_body,
        out_shape=jax.ShapeDtypeStruct((NPAD_DEG // 128, 128), jnp.float32),
    )(cnt3d)


def _l1_body(x_ref, w_ref, dinv_ref, out_ref):
    h = jnp.dot(x_ref[...], w_ref[0], preferred_element_type=jnp.float32)
    out_ref[...] = dinv_ref[...] * h


def _layer1(x, w_split, dinv_col, hd):
    in_dim = x.shape[1]
    return pl.pallas_call(
        _l1_body,
        grid=(NC, NBLKS),
        in_specs=[
            pl.BlockSpec((NB_ROWS, in_dim), lambda c, i: (i, 0)),
            pl.BlockSpec((1, in_dim, hd), lambda c, i: (c, 0, 0)),
            pl.BlockSpec((NB_ROWS, 1), lambda c, i: (i, 0)),
        ],
        out_specs=pl.BlockSpec((NB_ROWS, hd), lambda c, i: (c * NBLKS + i, 0)),
        out_shape=jax.ShapeDtypeStruct((NC * N_NODES, hd), jnp.float32),
    )(x, w_split, dinv_col)


def _mid_body(acc_lo, acc_hi, hp_lo, hp_hi, dinv_ref, b_ref, w_ref, out_ref):
    x = jnp.concatenate([acc_lo[...] + hp_lo[...], acc_hi[...] + hp_hi[...]],
                        axis=1)
    x = jnp.maximum(dinv_ref[...] * x + b_ref[...], 0.0)
    h = jnp.dot(x, w_ref[0], preferred_element_type=jnp.float32)
    out_ref[...] = dinv_ref[...] * h


def _layer_mid(acc_pad, hp, dinv_col, b_row, w_split, hd_in, hd_out):
    acc_lo = pl.BlockSpec((NB_ROWS, hd_in), lambda c, i: (i, 0))
    acc_hi = pl.BlockSpec((NB_ROWS, hd_in), lambda c, i: (HI_OFF + i, 0))
    hp_lo = pl.BlockSpec((NB_ROWS, hd_in), lambda c, i: (i, 0))
    hp_hi = pl.BlockSpec((NB_ROWS, hd_in), lambda c, i: (NBLKS + i, 0))
    return pl.pallas_call(
        _mid_body,
        grid=(NC, NBLKS),
        in_specs=[
            acc_lo, acc_hi, hp_lo, hp_hi,
            pl.BlockSpec((NB_ROWS, 1), lambda c, i: (i, 0)),
            pl.BlockSpec((1, 2 * hd_in), lambda c, i: (0, 0)),
            pl.BlockSpec((1, 2 * hd_in, hd_out), lambda c, i: (c, 0, 0)),
        ],
        out_specs=pl.BlockSpec((NB_ROWS, hd_out),
                               lambda c, i: (c * NBLKS + i, 0)),
        out_shape=jax.ShapeDtypeStruct((NC * N_NODES, hd_out), jnp.float32),
    )(acc_pad, acc_pad, hp, hp, dinv_col, b_row, w_split)


def _mid1_body(acc_lo, acc_hi, hp_lo, hp_hi, dinv_ref, b_ref, w_ref, out_ref):
    x = jnp.concatenate([acc_lo[...] + hp_lo[...], acc_hi[...] + hp_hi[...]],
                        axis=1)
    x = jnp.maximum(dinv_ref[...] * x + b_ref[...], 0.0)
    h = jnp.dot(x, w_ref[...], preferred_element_type=jnp.float32)
    out_ref[...] = dinv_ref[...] * h


def _layer_mid_full(acc_pad, hp, dinv_col, b_row, w, hd_in, hd_out):
    acc_lo = pl.BlockSpec((NB_ROWS, hd_in), lambda i: (i, 0))
    acc_hi = pl.BlockSpec((NB_ROWS, hd_in), lambda i: (HI_OFF + i, 0))
    hp_lo = pl.BlockSpec((NB_ROWS, hd_in), lambda i: (i, 0))
    hp_hi = pl.BlockSpec((NB_ROWS, hd_in), lambda i: (NBLKS + i, 0))
    return pl.pallas_call(
        _mid1_body,
        grid=(NBLKS,),
        in_specs=[
            acc_lo, acc_hi, hp_lo, hp_hi,
            pl.BlockSpec((NB_ROWS, 1), lambda i: (i, 0)),
            pl.BlockSpec((1, 2 * hd_in), lambda i: (0, 0)),
            pl.BlockSpec((2 * hd_in, hd_out), lambda i: (0, 0)),
        ],
        out_specs=pl.BlockSpec((NB_ROWS, hd_out), lambda i: (i, 0)),
        out_shape=jax.ShapeDtypeStruct((N_NODES, hd_out), jnp.float32),
    )(acc_pad, acc_pad, hp, hp, dinv_col, b_row, w)


def _fin_body(acc_a, acc_b, hp_ref, dinv_ref, b_ref, out_ref):
    x = acc_a[...] + acc_b[...] + hp_ref[...]
    out_ref[...] = dinv_ref[...] * x + b_ref[...]


def _layer_fin(acc_pad, hp, dinv_col, b_row, hd):
    full = pl.BlockSpec((NB_ROWS, hd), lambda i: (i, 0))
    acc_hi = pl.BlockSpec((NB_ROWS, hd), lambda i: (HI_OFF + i, 0))
    return pl.pallas_call(
        _fin_body,
        grid=(NBLKS,),
        in_specs=[
            full, acc_hi, full,
            pl.BlockSpec((NB_ROWS, 1), lambda i: (i, 0)),
            pl.BlockSpec((1, hd), lambda i: (0, 0)),
        ],
        out_specs=full,
        out_shape=jax.ShapeDtypeStruct((N_NODES, hd), jnp.float32),
    )(acc_pad, acc_pad, hp, dinv_col, b_row)


def kernel(edge_index, node_emb, W1, b1, W2, b2, W3, b3):
    src = edge_index[0].astype(jnp.int32)
    dst = edge_index[1].astype(jnp.int32)
    src2d = src.reshape(NBLK, EPB)
    srchi2d = (src + N_NODES).reshape(NBLK, EPB)
    dst2d = dst.reshape(NBLK, EPB)

    cnt_pad = _make_deg_kernel()(dst2d)
    dinv_col = _dinv_tc(cnt_pad).reshape(-1)[:N_NODES].reshape(N_NODES, 1)

    agg_split = _make_agg_split_kernel()
    agg_edges = _make_agg_edges_kernel()

    def split_cols(w, hd):
        return jnp.stack([w[:, :hd], w[:, hd:]])

    hp1 = _layer1(node_emb, split_cols(W1, 128), dinv_col, 128)
    a1 = agg_split(hp1, src2d, srchi2d, dst2d)
    hp2 = _layer_mid(a1, hp1, dinv_col, b1.reshape(1, -1),
                     split_cols(W2, 128), 128, 128)
    a2 = agg_split(hp2, src2d, srchi2d, dst2d)
    hp3 = _layer_mid_full(a2, hp2, dinv_col, b2.reshape(1, -1),
                          W3, 128, 128)
    a3 = agg_edges(hp3, src2d, dst2d)
    return _layer_fin(a3, hp3, dinv_col, b3.reshape(1, -1), 128)
